# baseline (speedup 1.0000x reference)
import functools

import numpy as np
import jax
import jax.numpy as jnp
from jax import lax
from jax.experimental import pallas as pl
from jax.experimental.pallas import tpu as pltpu
from jax.experimental.pallas import tpu_sc as plsc


def _gen_base_anchors():
    base_size = 16
    ratios = np.array([0.5, 1.0, 2.0], dtype=np.float64)
    scales = np.array([8.0, 16.0, 32.0], dtype=np.float64)
    base = np.array([0.0, 0.0, base_size - 1.0, base_size - 1.0])
    w = base[2] - base[0] + 1.0
    h = base[3] - base[1] + 1.0
    x_ctr = base[0] + 0.5 * (w - 1.0)
    y_ctr = base[1] + 0.5 * (h - 1.0)
    size = w * h
    ws_r = np.round(np.sqrt(size / ratios))
    hs_r = np.round(ws_r * ratios)
    anchors = []
    for i in range(3):
        w_i, h_i = ws_r[i], hs_r[i]
        for s in scales:
            ws, hs = w_i * s, h_i * s
            anchors.append([x_ctr - 0.5 * (ws - 1.0), y_ctr - 0.5 * (hs - 1.0),
                            x_ctr + 0.5 * (ws - 1.0), y_ctr + 0.5 * (hs - 1.0)])
    return np.array(anchors, dtype=np.float32)


_ANCH = _gen_base_anchors()
_AW = [float(a[2] - a[0] + 1.0) for a in _ANCH]
_AH = [float(a[3] - a[1] + 1.0) for a in _ANCH]
_ACX = [float(a[0] + 0.5 * (a[2] - a[0] + 1.0)) for a in _ANCH]
_ACY = [float(a[1] + 0.5 * (a[3] - a[1] + 1.0)) for a in _ANCH]

FEAT_STRIDE = 16
B, A, H, W = 16, 9, 64, 64
K = H * W
C4 = 4 * A
NC, NS, L = 2, 16, 16
HALF_ROWS = H // NC
ROWS = 8
CHUNK = ROWS * W
NCHUNK = HALF_ROWS // ROWS
GROUPS = CHUNK // L
SSTRIPE = CHUNK * A
NSTRIPE = K // CHUNK
BGROUP = B // 2

_mesh = plsc.VectorSubcoreMesh(
    core_axis_name="c", subcore_axis_name="s", num_cores=NC, num_subcores=NS)


@functools.partial(
    pl.kernel,
    out_type=(jax.ShapeDtypeStruct((B * K * A * 4,), jnp.float32),
              jax.ShapeDtypeStruct((B, K * A), jnp.float32)),
    mesh=_mesh,
    scratch_types=[
        pltpu.VMEM((C4, ROWS, W), jnp.float32),
        pltpu.VMEM((CHUNK * A * 4,), jnp.float32),
        pltpu.VMEM((CHUNK * A * 4,), jnp.float32),
        pltpu.VMEM((A, ROWS, W), jnp.float32),
        pltpu.VMEM((A, ROWS, W), jnp.float32),
        pltpu.SemaphoreType.DMA,
        pltpu.SemaphoreType.DMA,
    ],
    compiler_params=pltpu.CompilerParams(needs_layout_passes=False,
                                         use_tc_tiling_on_sc=True),
)
def _proposal_sc(bbox_hbm, cls_hbm, props_hbm, scores_hbm,
                 bb, po0, po1, cl0, cl1, in_sem, out_sem):
    sub = lax.axis_index("s")
    core = lax.axis_index("c")

    iota = lax.iota(jnp.int32, L)
    sx_lane = (iota * FEAT_STRIDE).astype(jnp.float32)
    idx9 = iota * A

    pos = (po0, po1)
    N_LIGHT, N_HEAVY = 3, 5

    def in_slice(gc):
        return bbox_hbm.at[gc >> 3, :,
                           pl.ds(pl.multiple_of((gc & 7) * ROWS, 8), ROWS), :]

    def run_chunks(start, count, bbs):
        nb = len(bbs)
        in_cp = {}
        if nb > 1:
            in_cp[0] = pltpu.async_copy(in_slice(start), bbs[0], in_sem)
        out_cp = {}
        for i in range(count):
            gc = start + i
            batch_d = gc >> 3
            if nb > 1:
                if i + 1 < count:
                    in_cp[i + 1] = pltpu.async_copy(
                        in_slice(gc + 1), bbs[(i + 1) % nb], in_sem)
                in_cp[i].wait()
            else:
                pltpu.sync_copy(in_slice(gc), bbs[0])
            if i >= 2:
                out_cp[i - 2].wait()
            bb = bbs[i % nb]
            po = pos[i % 2]

            hrow = (gc & 7) * ROWS

            def group_body(g2, carry, hrow=hrow, po=po, bb=bb):
                for gg in range(4):
                    g = g2 * 4 + gg
                    r = g // 4
                    wcol = (g % 4) * L
                    sx = sx_lane + (wcol * FEAT_STRIDE).astype(jnp.float32)
                    sy = ((hrow + r) * FEAT_STRIDE).astype(jnp.float32)
                    kk0 = idx9 + ((r * W + wcol) * A)
                    for a in range(A):
                        dx = bb[4 * a + 0, r, pl.ds(wcol, L)]
                        dy = bb[4 * a + 1, r, pl.ds(wcol, L)]
                        dw = bb[4 * a + 2, r, pl.ds(wcol, L)]
                        dh = bb[4 * a + 3, r, pl.ds(wcol, L)]
                        px = dx * _AW[a] + (sx + _ACX[a])
                        py = dy * _AH[a] + (sy + _ACY[a])
                        hw = jnp.exp(dw) * (0.5 * _AW[a])
                        hh = jnp.exp(dh) * (0.5 * _AH[a])
                        kkv = kk0 + a
                        fl = ((kkv >> 7) << 9) + (kkv & 127)
                        plsc.store_scatter(po, [fl], px - hw)
                        plsc.store_scatter(po, [fl + 128], py - hh)
                        plsc.store_scatter(po, [fl + 256], px + hw)
                        plsc.store_scatter(po, [fl + 384], py + hh)
                return carry

            lax.fori_loop(0, GROUPS // 4, group_body, 0)

            out_cp[i] = pltpu.async_copy(
                po,
                props_hbm.at[pl.ds(
                    pl.multiple_of(batch_d * (K * A * 4)
                                   + (gc & 7) * (CHUNK * A * 4), 128),
                    CHUNK * A * 4)],
                out_sem)
        for i in range(max(count - 2, 0), count):
            out_cp[i].wait()

    @pl.when(sub < NSTRIPE)
    def _light_path():
        hrow = sub * ROWS
        cls_bufs = (cl0, cl1)

        def cl_slice(b):
            return cls_hbm.at[core * BGROUP + b, pl.ds(A, A),
                              pl.ds(hrow, ROWS), :]

        ccp = {0: pltpu.async_copy(cl_slice(0), cl0, in_sem),
               1: pltpu.async_copy(cl_slice(1), cl1, in_sem)}
        run_chunks((sub * NC + core) * N_LIGHT, N_LIGHT, (bb,))

        def _score_body(sb):
            for b in range(BGROUP):
                ccp[b].wait()
                cl = cls_bufs[b % 2]
                bcol = jnp.full((L,), b, jnp.int32)

                def sgroup_body(g, carry, cl=cl, bcol=bcol, sb=sb):
                    r = g // 4
                    wcol = (g % 4) * L
                    colv = idx9 + ((r * W + wcol) * A)
                    for a in range(A):
                        plsc.store_scatter(sb, [bcol, colv + a],
                                           cl[a, r, pl.ds(wcol, L)])
                    return carry

                lax.fori_loop(0, GROUPS, sgroup_body, 0)
                if b + 2 < BGROUP:
                    ccp[b + 2] = pltpu.async_copy(
                        cl_slice(b + 2), cls_bufs[b % 2], in_sem)

            pltpu.sync_copy(
                sb,
                scores_hbm.at[pl.ds(core * BGROUP, BGROUP),
                              pl.ds(sub * SSTRIPE, SSTRIPE)])

        pl.run_scoped(_score_body,
                      pltpu.VMEM((BGROUP, SSTRIPE), jnp.float32))

    @pl.when(sub >= NSTRIPE)
    def _heavy_path():
        def _heavy_body(bb1):
            run_chunks(NS * N_LIGHT
                       + ((sub - NSTRIPE) * NC + core) * N_HEAVY, N_HEAVY,
                       (bb, bb1))

        pl.run_scoped(_heavy_body, pltpu.VMEM((C4, ROWS, W), jnp.float32))


def kernel(rpn_cls_probs, rpn_pred_bboxes, im_shapes, cfg_key):
    del im_shapes, cfg_key
    props_flat, scores = _proposal_sc(rpn_pred_bboxes, rpn_cls_probs)
    props = (props_flat.reshape(B, K * A // 128, 4, 128)
             .transpose(0, 1, 3, 2).reshape(B, K * A, 4))
    return props, scores


# speedup vs baseline: 1.0574x; 1.0574x over previous
import functools

import numpy as np
import jax
import jax.numpy as jnp
from jax import lax
from jax.experimental import pallas as pl
from jax.experimental.pallas import tpu as pltpu
from jax.experimental.pallas import tpu_sc as plsc


def _gen_base_anchors():
    base_size = 16
    ratios = np.array([0.5, 1.0, 2.0], dtype=np.float64)
    scales = np.array([8.0, 16.0, 32.0], dtype=np.float64)
    base = np.array([0.0, 0.0, base_size - 1.0, base_size - 1.0])
    w = base[2] - base[0] + 1.0
    h = base[3] - base[1] + 1.0
    x_ctr = base[0] + 0.5 * (w - 1.0)
    y_ctr = base[1] + 0.5 * (h - 1.0)
    size = w * h
    ws_r = np.round(np.sqrt(size / ratios))
    hs_r = np.round(ws_r * ratios)
    anchors = []
    for i in range(3):
        w_i, h_i = ws_r[i], hs_r[i]
        for s in scales:
            ws, hs = w_i * s, h_i * s
            anchors.append([x_ctr - 0.5 * (ws - 1.0), y_ctr - 0.5 * (hs - 1.0),
                            x_ctr + 0.5 * (ws - 1.0), y_ctr + 0.5 * (hs - 1.0)])
    return np.array(anchors, dtype=np.float32)


_ANCH = _gen_base_anchors()
_AW = [float(a[2] - a[0] + 1.0) for a in _ANCH]
_AH = [float(a[3] - a[1] + 1.0) for a in _ANCH]
_ACX = [float(a[0] + 0.5 * (a[2] - a[0] + 1.0)) for a in _ANCH]
_ACY = [float(a[1] + 0.5 * (a[3] - a[1] + 1.0)) for a in _ANCH]

FEAT_STRIDE = 16
B, A, H, W = 16, 9, 64, 64
K = H * W
C4 = 4 * A
NC, NS, L = 2, 16, 16
HALF_ROWS = H // NC
ROWS = 8
CHUNK = ROWS * W
NCHUNK = HALF_ROWS // ROWS
GROUPS = CHUNK // L
SSTRIPE = CHUNK * A
NSTRIPE = K // CHUNK
BGROUP = B // 2

_mesh = plsc.VectorSubcoreMesh(
    core_axis_name="c", subcore_axis_name="s", num_cores=NC, num_subcores=NS)


@functools.partial(
    pl.kernel,
    out_type=(jax.ShapeDtypeStruct((B * K * A * 4,), jnp.float32),
              jax.ShapeDtypeStruct((B, K * A), jnp.float32)),
    mesh=_mesh,
    scratch_types=[
        pltpu.VMEM((C4, ROWS, W), jnp.float32),
        pltpu.VMEM((CHUNK * A * 4,), jnp.float32),
        pltpu.VMEM((CHUNK * A * 4,), jnp.float32),
        pltpu.VMEM((A, ROWS, W), jnp.float32),
        pltpu.VMEM((A, ROWS, W), jnp.float32),
        pltpu.SemaphoreType.DMA,
        pltpu.SemaphoreType.DMA,
    ],
    compiler_params=pltpu.CompilerParams(needs_layout_passes=False,
                                         use_tc_tiling_on_sc=True),
)
def _proposal_sc(bbox_hbm, cls_hbm, props_hbm, scores_hbm,
                 bb, po0, po1, cl0, cl1, in_sem, out_sem):
    sub = lax.axis_index("s")
    core = lax.axis_index("c")

    iota = lax.iota(jnp.int32, L)
    sx_lane = (iota * FEAT_STRIDE).astype(jnp.float32)
    idx9 = iota * A

    pos = (po0, po1)
    N_LIGHT, N_HEAVY = 3, 5

    def in_slice(gc):
        return bbox_hbm.at[gc >> 3, :,
                           pl.ds(pl.multiple_of((gc & 7) * ROWS, 8), ROWS), :]

    def run_chunks(start, count, bbs):
        nb = len(bbs)
        in_cp = {}
        if nb > 1:
            in_cp[0] = pltpu.async_copy(in_slice(start), bbs[0], in_sem)
        out_cp = {}
        for i in range(count):
            gc = start + i
            batch_d = gc >> 3
            if nb > 1:
                if i + 1 < count:
                    in_cp[i + 1] = pltpu.async_copy(
                        in_slice(gc + 1), bbs[(i + 1) % nb], in_sem)
                in_cp[i].wait()
            else:
                pltpu.sync_copy(in_slice(gc), bbs[0])
            if i >= 2:
                out_cp[i - 2].wait()
            bb = bbs[i % nb]
            po = pos[i % 2]

            hrow = (gc & 7) * ROWS

            def group_body(g2, carry, hrow=hrow, po=po, bb=bb):
                for gg in range(2):
                    g = g2 * 2 + gg
                    r = g // 4
                    wcol = (g % 4) * L
                    sx = sx_lane + (wcol * FEAT_STRIDE).astype(jnp.float32)
                    sy = ((hrow + r) * FEAT_STRIDE).astype(jnp.float32)
                    kk0 = idx9 + ((r * W + wcol) * A)
                    for a in range(A):
                        dx = bb[4 * a + 0, r, pl.ds(wcol, L)]
                        dy = bb[4 * a + 1, r, pl.ds(wcol, L)]
                        dw = bb[4 * a + 2, r, pl.ds(wcol, L)]
                        dh = bb[4 * a + 3, r, pl.ds(wcol, L)]
                        px = dx * _AW[a] + (sx + _ACX[a])
                        py = dy * _AH[a] + (sy + _ACY[a])
                        hw = jnp.exp(dw) * (0.5 * _AW[a])
                        hh = jnp.exp(dh) * (0.5 * _AH[a])
                        kkv = kk0 + a
                        fl = ((kkv >> 7) << 9) + (kkv & 127)
                        plsc.store_scatter(po, [fl], px - hw)
                        plsc.store_scatter(po, [fl + 128], py - hh)
                        plsc.store_scatter(po, [fl + 256], px + hw)
                        plsc.store_scatter(po, [fl + 384], py + hh)
                return carry

            lax.fori_loop(0, GROUPS // 2, group_body, 0)

            out_cp[i] = pltpu.async_copy(
                po,
                props_hbm.at[pl.ds(
                    pl.multiple_of(batch_d * (K * A * 4)
                                   + (gc & 7) * (CHUNK * A * 4), 128),
                    CHUNK * A * 4)],
                out_sem)
        for i in range(max(count - 2, 0), count):
            out_cp[i].wait()

    @pl.when(sub < NSTRIPE)
    def _light_path():
        hrow = sub * ROWS
        cls_bufs = (cl0, cl1)

        def cl_slice(b):
            return cls_hbm.at[core * BGROUP + b, pl.ds(A, A),
                              pl.ds(hrow, ROWS), :]

        ccp = {0: pltpu.async_copy(cl_slice(0), cl0, in_sem),
               1: pltpu.async_copy(cl_slice(1), cl1, in_sem)}
        run_chunks((sub * NC + core) * N_LIGHT, N_LIGHT, (bb,))

        def _score_body(sb):
            for b in range(BGROUP):
                ccp[b].wait()
                cl = cls_bufs[b % 2]
                bcol = jnp.full((L,), b, jnp.int32)

                def sgroup_body(g, carry, cl=cl, bcol=bcol, sb=sb):
                    r = g // 4
                    wcol = (g % 4) * L
                    colv = idx9 + ((r * W + wcol) * A)
                    for a in range(A):
                        plsc.store_scatter(sb, [bcol, colv + a],
                                           cl[a, r, pl.ds(wcol, L)])
                    return carry

                lax.fori_loop(0, GROUPS, sgroup_body, 0)
                if b + 2 < BGROUP:
                    ccp[b + 2] = pltpu.async_copy(
                        cl_slice(b + 2), cls_bufs[b % 2], in_sem)

            pltpu.sync_copy(
                sb,
                scores_hbm.at[pl.ds(core * BGROUP, BGROUP),
                              pl.ds(sub * SSTRIPE, SSTRIPE)])

        pl.run_scoped(_score_body,
                      pltpu.VMEM((BGROUP, SSTRIPE), jnp.float32))

    @pl.when(sub >= NSTRIPE)
    def _heavy_path():
        def _heavy_body(bb1):
            run_chunks(NS * N_LIGHT
                       + ((sub - NSTRIPE) * NC + core) * N_HEAVY, N_HEAVY,
                       (bb, bb1))

        pl.run_scoped(_heavy_body, pltpu.VMEM((C4, ROWS, W), jnp.float32))


def kernel(rpn_cls_probs, rpn_pred_bboxes, im_shapes, cfg_key):
    del im_shapes, cfg_key
    props_flat, scores = _proposal_sc(rpn_pred_bboxes, rpn_cls_probs)
    props = (props_flat.reshape(B, K * A // 128, 4, 128)
             .transpose(0, 1, 3, 2).reshape(B, K * A, 4))
    return props, scores
